# baseline (speedup 1.0000x reference)
import jax
import jax.numpy as jnp
from jax.experimental import pallas as pl
from jax.experimental.pallas import tpu as pltpu

H = 1024
B = 32
ENC = 2048
BK = 32
NBLK = ENC // (2 * BK)
NA = 6
NC = 6
HH = H // 2


def _fused_kernel(x_ref, hu_ref, he_ref,
                  wihu_ref, whhu_ref, bihu_ref, bhhu_ref,
                  ka_ref, va_ref, kb_ref, vb_ref,
                  wihe_ref, whhe_ref, bihe_ref, bhhe_ref,
                  o_ref,
                  ugi_ref, ugh_ref, q_ref, m_ref, l_ref, acc_ref,
                  ein_ref, egi_ref, egh_ref):
    t = pl.program_id(0)

    def mmT(a, w):
        return jax.lax.dot_general(
            a, w, (((1,), (1,)), ((), ())),
            preferred_element_type=jnp.float32)

    @pl.when(t < NA)
    def _():
        ugi_ref[t] = mmT(x_ref[...], wihu_ref[0]) + bihu_ref[t]
        ugh_ref[t] = mmT(hu_ref[...], whhu_ref[0]) + bhhu_ref[t]

    @pl.when(t == NA - 1)
    def _():
        def ugate(ref, g):
            return jnp.concatenate([ref[2 * g], ref[2 * g + 1]], axis=1)
        r = jax.nn.sigmoid(ugate(ugi_ref, 0) + ugate(ugh_ref, 0))
        z = jax.nn.sigmoid(ugate(ugi_ref, 1) + ugate(ugh_ref, 1))
        n = jnp.tanh(ugate(ugi_ref, 2) + r * ugate(ugh_ref, 2))
        q_ref[...] = (1.0 - z) * n + z * hu_ref[...]
        m_ref[...] = jnp.full_like(m_ref, -jnp.inf)
        l_ref[...] = jnp.zeros_like(l_ref)
        acc_ref[...] = jnp.zeros_like(acc_ref)

    @pl.when(jnp.logical_and(t >= NA, t < NA + NBLK))
    def _():
        q = q_ref[...]

        def upd(k, v):
            s = jnp.sum(q[:, None, :] * k, axis=2)
            m_prev = m_ref[...]
            s_max = jnp.max(s, axis=1, keepdims=True)
            m_new = jnp.maximum(m_prev, s_max)
            alpha = jnp.exp(m_prev - m_new)
            p = jnp.exp(s - m_new[:, :1])
            l_ref[...] = l_ref[...] * alpha + jnp.sum(p, axis=1, keepdims=True)
            acc_ref[...] = acc_ref[...] * alpha[:, :1] + jnp.sum(p[:, :, None] * v, axis=1)
            m_ref[...] = m_new

        upd(ka_ref[...], va_ref[...])
        upd(kb_ref[...], vb_ref[...])

    @pl.when(t == NA + NBLK - 1)
    def _():
        ein_ref[:, :H] = acc_ref[...] / l_ref[:, :1]
        ein_ref[:, H:] = x_ref[...]

    @pl.when(t >= NA + NBLK)
    def _():
        tc = t - (NA + NBLK)
        egi_ref[tc] = mmT(ein_ref[...], wihe_ref[0]) + bihe_ref[tc]
        egh_ref[tc] = mmT(he_ref[...], whhe_ref[0]) + bhhe_ref[tc]

    @pl.when(t == NA + NBLK + NC - 1)
    def _():
        def gate(ref, g):
            return jnp.concatenate([ref[2 * g], ref[2 * g + 1]], axis=1)
        r = jax.nn.sigmoid(gate(egi_ref, 0) + gate(egh_ref, 0))
        z = jax.nn.sigmoid(gate(egi_ref, 1) + gate(egh_ref, 1))
        n = jnp.tanh(gate(egi_ref, 2) + r * gate(egh_ref, 2))
        o_ref[...] = (1.0 - z) * n + z * he_ref[...]


def kernel(embedded, ponder_decoder_hidden, attn_keys, attn_vals,
           W_ih_u, W_hh_u, b_ih_u, b_hh_u,
           W_ih_e, W_hh_e, b_ih_e, b_hh_e):
    x = embedded[:, 0, :]
    h_u = ponder_decoder_hidden[0, :, :H]
    h_e = ponder_decoder_hidden[0, :, H:]

    wihu = W_ih_u.reshape(6, HH, H)
    whhu = W_hh_u.reshape(6, HH, H)
    bihu = b_ih_u.reshape(6, 1, HH)
    bhhu = b_hh_u.reshape(6, 1, HH)
    wihe = W_ih_e.reshape(6, HH, 2 * H)
    whhe = W_hh_e.reshape(6, HH, H)
    bihe = b_ih_e.reshape(6, 1, HH)
    bhhe = b_hh_e.reshape(6, 1, HH)

    T = NA + NBLK + NC

    def aidx(t):
        return jnp.minimum(t, NA - 1)

    def bidx(t):
        return jnp.clip(t - NA, 0, NBLK - 1)

    def cidx(t):
        return jnp.clip(t - (NA + NBLK), 0, NC - 1)

    out = pl.pallas_call(
        _fused_kernel,
        grid=(T,),
        in_specs=[
            pl.BlockSpec((B, H), lambda t: (0, 0)),
            pl.BlockSpec((B, H), lambda t: (0, 0)),
            pl.BlockSpec((B, H), lambda t: (0, 0)),
            pl.BlockSpec((1, HH, H), lambda t: (aidx(t), 0, 0)),
            pl.BlockSpec((1, HH, H), lambda t: (aidx(t), 0, 0)),
            pl.BlockSpec((6, 1, HH), lambda t: (0, 0, 0)),
            pl.BlockSpec((6, 1, HH), lambda t: (0, 0, 0)),
            pl.BlockSpec((B, BK, H), lambda t: (0, 2 * bidx(t), 0)),
            pl.BlockSpec((B, BK, H), lambda t: (0, 2 * bidx(t), 0)),
            pl.BlockSpec((B, BK, H), lambda t: (0, 2 * bidx(t) + 1, 0)),
            pl.BlockSpec((B, BK, H), lambda t: (0, 2 * bidx(t) + 1, 0)),
            pl.BlockSpec((1, HH, 2 * H), lambda t: (cidx(t), 0, 0)),
            pl.BlockSpec((1, HH, H), lambda t: (cidx(t), 0, 0)),
            pl.BlockSpec((6, 1, HH), lambda t: (0, 0, 0)),
            pl.BlockSpec((6, 1, HH), lambda t: (0, 0, 0)),
        ],
        out_specs=pl.BlockSpec((B, H), lambda t: (0, 0)),
        out_shape=jax.ShapeDtypeStruct((B, H), jnp.float32),
        compiler_params=pltpu.CompilerParams(
            vmem_limit_bytes=100 * 1024 * 1024),
        scratch_shapes=[
            pltpu.VMEM((6, B, HH), jnp.float32),
            pltpu.VMEM((6, B, HH), jnp.float32),
            pltpu.VMEM((B, H), jnp.float32),
            pltpu.VMEM((B, 128), jnp.float32),
            pltpu.VMEM((B, 128), jnp.float32),
            pltpu.VMEM((B, H), jnp.float32),
            pltpu.VMEM((B, 2 * H), jnp.float32),
            pltpu.VMEM((6, B, HH), jnp.float32),
            pltpu.VMEM((6, B, HH), jnp.float32),
        ],
    )(x, h_u, h_e, wihu, whhu, bihu, bhhu,
      attn_keys, attn_vals, attn_keys, attn_vals,
      wihe, whhe, bihe, bhhe)
    return out[:, None, :]
